# baseline (speedup 1.0000x reference)
import functools

import jax
import jax.numpy as jnp
from jax import lax
from jax.experimental import pallas as pl
from jax.experimental.pallas import tpu as pltpu
from jax.experimental.pallas import tpu_sc as plsc


_EPS = 1e-12
_TOK_BLOCK = 512


def _argmin_body(lhs_ref, en_ref, esq_ref, zsq_ref, idx_ref, lsum_ref):
    k = en_ref.shape[0]
    h = k // 2
    lhs = lhs_ref[...]
    zsq = zsq_ref[...]

    def half(lo):
        dot = lax.dot_general(lhs, en_ref[lo:lo + h, :],
                              (((1,), (1,)), ((), ())),
                              preferred_element_type=jnp.float32)
        d = (zsq + esq_ref[:, lo:lo + h]) - dot
        m = jnp.min(d, axis=1, keepdims=True)
        i = jnp.argmin(d, axis=1, keepdims=True).astype(jnp.int32)
        return m, i

    m1, i1 = half(0)
    m2, i2 = half(h)
    a1 = m1.astype(jnp.bfloat16).astype(jnp.float32)
    win2 = m2 < a1
    idx_ref[...] = jnp.where(win2, i2 + h, i1)
    lsum_ref[...] = jnp.sum(jnp.where(win2, m2, m1)).reshape(1, 1, 1)


def _tc_argmin(lhs_bf, en_bf, esq_row, zsq_col):
    bn, c = lhs_bf.shape
    k = en_bf.shape[0]
    t = _TOK_BLOCK
    nb = bn // t
    return pl.pallas_call(
        _argmin_body,
        grid=(nb,),
        in_specs=[
            pl.BlockSpec((t, c), lambda i: (i, 0)),
            pl.BlockSpec((k, c), lambda i: (0, 0)),
            pl.BlockSpec((1, k), lambda i: (0, 0)),
            pl.BlockSpec((t, 1), lambda i: (i, 0)),
        ],
        out_specs=[
            pl.BlockSpec((t, 1), lambda i: (i, 0)),
            pl.BlockSpec((1, 1, 1), lambda i: (i, 0, 0)),
        ],
        out_shape=[
            jax.ShapeDtypeStruct((bn, 1), jnp.int32),
            jax.ShapeDtypeStruct((nb, 1, 1), jnp.float32),
        ],
        compiler_params=pltpu.CompilerParams(
            dimension_semantics=("parallel",),
        ),
    )(lhs_bf, en_bf, esq_row, zsq_col)


def _sc_gather(en, idx):
    bn = idx.shape[0]
    k, c = en.shape
    nw = 32
    b_per_w = bn // nw
    mesh = plsc.VectorSubcoreMesh(core_axis_name="c", subcore_axis_name="s")

    @functools.partial(
        pl.kernel,
        mesh=mesh,
        out_type=jax.ShapeDtypeStruct((bn, c), jnp.float32),
        scratch_types=[
            pltpu.VMEM((b_per_w,), jnp.int32),
            pltpu.VMEM((b_per_w, c), jnp.float32),
            pltpu.SemaphoreType.DMA,
        ],
        compiler_params=pltpu.CompilerParams(use_tc_tiling_on_sc=False),
    )
    def gather_kernel(en_hbm, idx_hbm, out_hbm, idx_v, rows_v, sem):
        wid = lax.axis_index("s") * 2 + lax.axis_index("c")
        base = wid * b_per_w
        pltpu.sync_copy(idx_hbm.at[pl.ds(base, b_per_w)], idx_v)
        pltpu.async_copy(en_hbm.at[idx_v], rows_v, sem).wait()
        pltpu.sync_copy(rows_v, out_hbm.at[pl.ds(base, b_per_w)])

    return gather_kernel(en, idx)


def _l2norm(t):
    n = jnp.sqrt(jnp.sum(t * t, axis=-1, keepdims=True))
    return t / jnp.maximum(n, _EPS)


def kernel(z, codebook):
    b, n, c = z.shape
    k = codebook.shape[0]

    z_n = _l2norm(z)
    e_n = _l2norm(codebook)
    flat = z_n.reshape(b * n, c)
    lhs_bf = (2.0 * flat).astype(jnp.bfloat16)
    en_bf = e_n.astype(jnp.bfloat16)
    zsq_col = jnp.sum(flat ** 2, axis=1, keepdims=True)
    esq_row = jnp.sum(e_n ** 2, axis=1)[None, :]

    idx_col, lsum = _tc_argmin(lhs_bf, en_bf, esq_row, zsq_col)
    idx = idx_col.reshape(b * n)

    zq_flat = _sc_gather(e_n, idx)

    z_q = zq_flat.reshape(b, n, c)
    loss = jnp.sum(lsum) / (b * n * c)
    embed_ind = idx.reshape(b, n)
    return z_q, loss, embed_ind


# speedup vs baseline: 1.0078x; 1.0078x over previous
import functools

import jax
import jax.numpy as jnp
from jax import lax
from jax.experimental import pallas as pl
from jax.experimental.pallas import tpu as pltpu
from jax.experimental.pallas import tpu_sc as plsc


_EPS = 1e-12
_TOK_BLOCK = 1024


def _argmin_body(lhs_ref, en_ref, esq_ref, zsq_ref, idx_ref, lsum_ref):
    k = en_ref.shape[0]
    h = k // 2
    lhs = lhs_ref[...]
    zsq = zsq_ref[...]

    def half(lo):
        dot = lax.dot_general(lhs, en_ref[lo:lo + h, :],
                              (((1,), (1,)), ((), ())),
                              preferred_element_type=jnp.float32)
        d = (zsq + esq_ref[:, lo:lo + h]) - dot
        m = jnp.min(d, axis=1, keepdims=True)
        i = jnp.argmin(d, axis=1, keepdims=True).astype(jnp.int32)
        return m, i

    m1, i1 = half(0)
    m2, i2 = half(h)
    a1 = m1.astype(jnp.bfloat16).astype(jnp.float32)
    win2 = m2 < a1
    idx_ref[...] = jnp.where(win2, i2 + h, i1)
    lsum_ref[...] = jnp.sum(jnp.where(win2, m2, m1)).reshape(1, 1, 1)


def _tc_argmin(lhs_bf, en_bf, esq_row, zsq_col):
    bn, c = lhs_bf.shape
    k = en_bf.shape[0]
    t = _TOK_BLOCK
    nb = bn // t
    return pl.pallas_call(
        _argmin_body,
        grid=(nb,),
        in_specs=[
            pl.BlockSpec((t, c), lambda i: (i, 0)),
            pl.BlockSpec((k, c), lambda i: (0, 0)),
            pl.BlockSpec((1, k), lambda i: (0, 0)),
            pl.BlockSpec((t, 1), lambda i: (i, 0)),
        ],
        out_specs=[
            pl.BlockSpec((t, 1), lambda i: (i, 0)),
            pl.BlockSpec((1, 1, 1), lambda i: (i, 0, 0)),
        ],
        out_shape=[
            jax.ShapeDtypeStruct((bn, 1), jnp.int32),
            jax.ShapeDtypeStruct((nb, 1, 1), jnp.float32),
        ],
        compiler_params=pltpu.CompilerParams(
            dimension_semantics=("parallel",),
        ),
    )(lhs_bf, en_bf, esq_row, zsq_col)


def _sc_gather(en, idx):
    bn = idx.shape[0]
    k, c = en.shape
    nw = 32
    b_per_w = bn // nw
    mesh = plsc.VectorSubcoreMesh(core_axis_name="c", subcore_axis_name="s")

    @functools.partial(
        pl.kernel,
        mesh=mesh,
        out_type=jax.ShapeDtypeStruct((bn, c), jnp.float32),
        scratch_types=[
            pltpu.VMEM((b_per_w,), jnp.int32),
            pltpu.VMEM((b_per_w, c), jnp.float32),
            pltpu.SemaphoreType.DMA,
        ],
        compiler_params=pltpu.CompilerParams(use_tc_tiling_on_sc=False),
    )
    def gather_kernel(en_hbm, idx_hbm, out_hbm, idx_v, rows_v, sem):
        wid = lax.axis_index("s") * 2 + lax.axis_index("c")
        base = wid * b_per_w
        pltpu.sync_copy(idx_hbm.at[pl.ds(base, b_per_w)], idx_v)
        pltpu.async_copy(en_hbm.at[idx_v], rows_v, sem).wait()
        pltpu.sync_copy(rows_v, out_hbm.at[pl.ds(base, b_per_w)])

    return gather_kernel(en, idx)


def _l2norm(t):
    n = jnp.sqrt(jnp.sum(t * t, axis=-1, keepdims=True))
    return t / jnp.maximum(n, _EPS)


def kernel(z, codebook):
    b, n, c = z.shape
    k = codebook.shape[0]

    z_n = _l2norm(z)
    e_n = _l2norm(codebook)
    flat = z_n.reshape(b * n, c)
    lhs_bf = (2.0 * flat).astype(jnp.bfloat16)
    en_bf = e_n.astype(jnp.bfloat16)
    zsq_col = jnp.sum(flat ** 2, axis=1, keepdims=True)
    esq_row = jnp.sum(e_n ** 2, axis=1)[None, :]

    idx_col, lsum = _tc_argmin(lhs_bf, en_bf, esq_row, zsq_col)
    idx = idx_col.reshape(b * n)

    zq_flat = _sc_gather(e_n, idx)

    z_q = zq_flat.reshape(b, n, c)
    loss = jnp.sum(lsum) / (b * n * c)
    embed_ind = idx.reshape(b, n)
    return z_q, loss, embed_ind
